# baseline (speedup 1.0000x reference)
import functools
import math

import jax
import jax.numpy as jnp
import numpy as np
from jax.experimental import pallas as pl
from jax.experimental.pallas import tpu as pltpu

B, H, T, DH = 1, 16, 2048, 64
NB_FEAT = int(DH * math.log(DH))
MPAD = 384
PRED_LEN = 128
NSEL = 4
TQ = 1024


def _dotbf(a, b, dims):
    return jax.lax.dot_general(
        a.astype(jnp.bfloat16), b.astype(jnp.bfloat16),
        (dims, ((), ())), preferred_element_type=jnp.float32)


def _fused_kernel(q_ref, k_ref, v_ref, w_ref, encw_ref, encb_ref,
                  lng_ref, lnb_ref, decw_ref, decb_ref, cexp_ref, o_ref):
    q = q_ref[0, 0]
    k = k_ref[0, 0]
    v = v_ref[0, 0]
    w = w_ref[...]
    f32 = jnp.float32
    bf16 = jnp.bfloat16
    qb, kb, vb = q.astype(bf16), k.astype(bf16), v.astype(bf16)

    data_norm = DH ** -0.25
    ratio = NB_FEAT ** -0.5
    feat_ok = jax.lax.broadcasted_iota(jnp.int32, (1, MPAD), 1) < NB_FEAT

    projq = data_norm * _dotbf(qb, w, ((1,), (1,)))
    projk = data_norm * _dotbf(kb, w, ((1,), (1,)))

    diag_q = (data_norm ** 2) * jnp.sum(q * q, axis=-1, keepdims=True) * 0.5
    diag_k = (data_norm ** 2) * jnp.sum(k * k, axis=-1, keepdims=True) * 0.5
    neg = jnp.float32(-jnp.inf)
    stab_q = jnp.max(jnp.where(feat_ok, projq, neg), axis=-1, keepdims=True)
    stab_k = jnp.max(jnp.where(feat_ok, projk, neg))

    pq = ratio * (jnp.exp(projq - diag_q - stab_q) + 1e-6)
    pq = jnp.where(feat_ok, pq, 0.0)
    pk = ratio * (jnp.exp(projk - diag_k - stab_k) + 1e-6)
    pk = jnp.where(feat_ok, pk, 0.0)

    pqb = pq.astype(bf16)
    kv = _dotbf(pk.astype(bf16), vb, ((0,), (0,)))
    num = _dotbf(pqb, kv, ((1,), (0,)))
    pk_sum = jnp.sum(pk, axis=0, keepdims=True)
    den = jnp.sum(pqb.astype(f32)
                  * pk_sum.astype(bf16).astype(f32),
                  axis=-1, keepdims=True)
    ctx = num / (den + 1e-6)

    encw = encw_ref[...]
    h = (_dotbf(vb, encw[0:DH], ((1,), (0,)))
         + _dotbf(ctx, encw[DH:2 * DH], ((1,), (0,)))
         + _dotbf(v * ctx, encw[2 * DH:3 * DH], ((1,), (0,)))
         + encb_ref[...])
    mu = jnp.mean(h, axis=-1, keepdims=True)
    var = jnp.mean((h - mu) * (h - mu), axis=-1, keepdims=True)
    h = (h - mu) * jax.lax.rsqrt(var + 1e-5) * lng_ref[...] + lnb_ref[...]
    h = jax.nn.gelu(h)
    lowres = _dotbf(h, decw_ref[...], ((1,), (0,))) + decb_ref[...]

    x = lowres
    m1 = jnp.max(x, axis=-1, keepdims=True)
    c1 = jnp.sum(jnp.where(x >= m1, 1.0, 0.0), axis=-1, keepdims=True)
    x = jnp.where(x >= m1, neg, x)
    m2 = jnp.max(x, axis=-1, keepdims=True)
    c2 = jnp.sum(jnp.where(x >= m2, 1.0, 0.0), axis=-1, keepdims=True) + c1
    x = jnp.where(x >= m2, neg, x)
    m3 = jnp.max(x, axis=-1, keepdims=True)
    c3 = jnp.sum(jnp.where(x >= m3, 1.0, 0.0), axis=-1, keepdims=True) + c2
    x = jnp.where(x >= m3, neg, x)
    m4 = jnp.max(x, axis=-1, keepdims=True)
    m = jnp.where(c1 >= NSEL, m1,
                  jnp.where(c2 >= NSEL, m2, jnp.where(c3 >= NSEL, m3, m4)))
    gm1 = jnp.where(lowres >= m, 0.0, -1.0).astype(bf16)

    cexp = cexp_ref[...]
    a_cat = jnp.concatenate([qb * jnp.bfloat16(DH ** -0.5), gm1], axis=1)
    k_cat = jnp.concatenate([kb, cexp], axis=1)
    for i in range(T // TQ):
        sl = slice(i * TQ, (i + 1) * TQ)
        s = jax.lax.dot_general(
            a_cat[sl], k_cat, (((1,), (1,)), ((), ())),
            preferred_element_type=f32)
        smax = jnp.max(s, axis=-1, keepdims=True)
        p = jnp.exp(s - smax)
        p = p * (1.0 / jnp.sum(p, axis=-1, keepdims=True))
        o_ref[0, 0, sl, :] = _dotbf(p, vb, ((1,), (0,)))


@functools.partial(jax.jit, static_argnames=("interpret",))
def kernel(q, k, v, W_perf, enc_w, enc_b, ln_g, ln_b, dec_w, dec_b,
           interpret=False):
    f32 = jnp.float32
    w_pad = jnp.zeros((MPAD, DH), f32).at[:NB_FEAT].set(W_perf)
    encb = enc_b.reshape(1, 2 * DH)
    lng = ln_g.reshape(1, 2 * DH)
    lnb = ln_b.reshape(1, 2 * DH)
    decb = dec_b.reshape(1, PRED_LEN)

    gid_np = (np.arange(T) * PRED_LEN) // T
    cexp = jnp.asarray(
        (gid_np[:, None] == np.arange(PRED_LEN)[None, :]) * float(2 ** 33),
        dtype=jnp.bfloat16)

    head_spec = pl.BlockSpec((1, 1, T, DH), lambda h: (0, h, 0, 0))
    full = lambda shape: pl.BlockSpec(shape, lambda h: tuple(0 for _ in shape))

    out = pl.pallas_call(
        _fused_kernel,
        grid=(H,),
        in_specs=[head_spec, head_spec, head_spec,
                  full((MPAD, DH)), full((3 * DH, 2 * DH)), full((1, 2 * DH)),
                  full((1, 2 * DH)), full((1, 2 * DH)),
                  full((2 * DH, PRED_LEN)), full((1, PRED_LEN)),
                  full((T, PRED_LEN))],
        out_specs=head_spec,
        out_shape=jax.ShapeDtypeStruct((B, H, T, DH), f32),
        compiler_params=pltpu.CompilerParams(
            dimension_semantics=("arbitrary",)),
        interpret=interpret,
    )(q, k, v, w_pad, enc_w, encb, lng, lnb, dec_w, decb, cexp)

    return out


# speedup vs baseline: 1.1225x; 1.1225x over previous
import functools
import math

import jax
import jax.numpy as jnp
import numpy as np
from jax.experimental import pallas as pl
from jax.experimental.pallas import tpu as pltpu

B, H, T, DH = 1, 16, 2048, 64
NB_FEAT = int(DH * math.log(DH))
MPAD = 384
PRED_LEN = 128
NSEL = 4
TQ = 512


def _dotbf(a, b, dims):
    return jax.lax.dot_general(
        a.astype(jnp.bfloat16), b.astype(jnp.bfloat16),
        (dims, ((), ())), preferred_element_type=jnp.float32)


def _fused_kernel(q_ref, k_ref, v_ref, w_ref, encw_ref, encb_ref,
                  lng_ref, lnb_ref, decw_ref, decb_ref, cexp_ref, o_ref):
    q = q_ref[0, 0]
    k = k_ref[0, 0]
    v = v_ref[0, 0]
    w = w_ref[...]
    f32 = jnp.float32
    bf16 = jnp.bfloat16
    qb, kb, vb = q.astype(bf16), k.astype(bf16), v.astype(bf16)

    data_norm = DH ** -0.25
    ratio = NB_FEAT ** -0.5
    feat_ok = jax.lax.broadcasted_iota(jnp.int32, (1, MPAD), 1) < NB_FEAT

    projq = data_norm * _dotbf(qb, w, ((1,), (1,)))
    projk = data_norm * _dotbf(kb, w, ((1,), (1,)))

    diag_q = (data_norm ** 2) * jnp.sum(q * q, axis=-1, keepdims=True) * 0.5
    diag_k = (data_norm ** 2) * jnp.sum(k * k, axis=-1, keepdims=True) * 0.5
    neg = jnp.float32(-jnp.inf)
    stab_q = jnp.max(jnp.where(feat_ok, projq, neg), axis=-1, keepdims=True)
    stab_k = jnp.max(jnp.where(feat_ok, projk, neg))

    pq = ratio * (jnp.exp(projq - diag_q - stab_q) + 1e-6)
    pq = jnp.where(feat_ok, pq, 0.0)
    pk = ratio * (jnp.exp(projk - diag_k - stab_k) + 1e-6)
    pk = jnp.where(feat_ok, pk, 0.0)

    pqb = pq.astype(bf16)
    kv = _dotbf(pk.astype(bf16), vb, ((0,), (0,)))
    num = _dotbf(pqb, kv, ((1,), (0,)))
    pk_sum = jnp.sum(pk, axis=0, keepdims=True)
    den = jnp.sum(pqb.astype(f32)
                  * pk_sum.astype(bf16).astype(f32),
                  axis=-1, keepdims=True)
    ctx = num / (den + 1e-6)

    encw = encw_ref[...]
    h = (_dotbf(vb, encw[0:DH], ((1,), (0,)))
         + _dotbf(ctx, encw[DH:2 * DH], ((1,), (0,)))
         + _dotbf(v * ctx, encw[2 * DH:3 * DH], ((1,), (0,)))
         + encb_ref[...])
    mu = jnp.mean(h, axis=-1, keepdims=True)
    var = jnp.mean((h - mu) * (h - mu), axis=-1, keepdims=True)
    h = (h - mu) * jax.lax.rsqrt(var + 1e-5) * lng_ref[...] + lnb_ref[...]
    h = jax.nn.gelu(h)
    lowres = _dotbf(h, decw_ref[...], ((1,), (0,))) + decb_ref[...]

    x = lowres
    m1 = jnp.max(x, axis=-1, keepdims=True)
    c1 = jnp.sum(jnp.where(x >= m1, 1.0, 0.0), axis=-1, keepdims=True)
    x = jnp.where(x >= m1, neg, x)
    m2 = jnp.max(x, axis=-1, keepdims=True)
    c2 = jnp.sum(jnp.where(x >= m2, 1.0, 0.0), axis=-1, keepdims=True) + c1
    x = jnp.where(x >= m2, neg, x)
    m3 = jnp.max(x, axis=-1, keepdims=True)
    c3 = jnp.sum(jnp.where(x >= m3, 1.0, 0.0), axis=-1, keepdims=True) + c2
    x = jnp.where(x >= m3, neg, x)
    m4 = jnp.max(x, axis=-1, keepdims=True)
    m = jnp.where(c1 >= NSEL, m1,
                  jnp.where(c2 >= NSEL, m2, jnp.where(c3 >= NSEL, m3, m4)))
    gm1 = jnp.where(lowres >= m, 0.0, -1.0).astype(bf16)

    cexp = cexp_ref[...]
    a_cat = jnp.concatenate([qb * jnp.bfloat16(DH ** -0.5), gm1], axis=1)
    k_cat = jnp.concatenate([kb, cexp], axis=1)
    for i in range(T // TQ):
        sl = slice(i * TQ, (i + 1) * TQ)
        s = jax.lax.dot_general(
            a_cat[sl], k_cat, (((1,), (1,)), ((), ())),
            preferred_element_type=f32)
        smax = jnp.max(s, axis=-1, keepdims=True)
        p = jnp.exp(s - smax)
        p = p * (1.0 / jnp.sum(p, axis=-1, keepdims=True))
        o_ref[0, 0, sl, :] = _dotbf(p, vb, ((1,), (0,)))


@functools.partial(jax.jit, static_argnames=("interpret",))
def kernel(q, k, v, W_perf, enc_w, enc_b, ln_g, ln_b, dec_w, dec_b,
           interpret=False):
    f32 = jnp.float32
    w_pad = jnp.zeros((MPAD, DH), f32).at[:NB_FEAT].set(W_perf)
    encb = enc_b.reshape(1, 2 * DH)
    lng = ln_g.reshape(1, 2 * DH)
    lnb = ln_b.reshape(1, 2 * DH)
    decb = dec_b.reshape(1, PRED_LEN)

    gid_np = (np.arange(T) * PRED_LEN) // T
    cexp = jnp.asarray(
        (gid_np[:, None] == np.arange(PRED_LEN)[None, :]) * float(2 ** 33),
        dtype=jnp.bfloat16)

    head_spec = pl.BlockSpec((1, 1, T, DH), lambda h: (0, h, 0, 0))
    full = lambda shape: pl.BlockSpec(shape, lambda h: tuple(0 for _ in shape))

    out = pl.pallas_call(
        _fused_kernel,
        grid=(H,),
        in_specs=[head_spec, head_spec, head_spec,
                  full((MPAD, DH)), full((3 * DH, 2 * DH)), full((1, 2 * DH)),
                  full((1, 2 * DH)), full((1, 2 * DH)),
                  full((2 * DH, PRED_LEN)), full((1, PRED_LEN)),
                  full((T, PRED_LEN))],
        out_specs=head_spec,
        out_shape=jax.ShapeDtypeStruct((B, H, T, DH), f32),
        compiler_params=pltpu.CompilerParams(
            dimension_semantics=("arbitrary",)),
        interpret=interpret,
    )(q, k, v, w_pad, enc_w, encb, lng, lnb, dec_w, decb, cexp)

    return out
